# baseline (speedup 1.0000x reference)
import jax
import jax.numpy as jnp
from jax import lax
from jax.experimental import pallas as pl
from jax.experimental.pallas import tpu as pltpu
from jax.experimental.pallas import tpu_sc as plsc

N = 50000
D = 64
HALF = 32
E = 800000
NSC = 2
NTILE = 16
EPAD = 819200
EPT = EPAD // NTILE
EB = 400
NP = 50048
ZR = NP // NTILE
CB = 184

MLP_BN = 1000


def _mlp_body(c_ref, t_ref, w1c_ref, w1t_ref, b1_ref, w2_ref, b2_ref, out_ref):
    x = c_ref[...] @ w1c_ref[...] + t_ref[...] @ w1t_ref[...] + b1_ref[...]
    x = jnp.maximum(x, 0.0)
    x = jnp.maximum(x @ w2_ref[...] + b2_ref[...], 0.0)
    ss = jnp.sum(x * x, axis=-1, keepdims=True)
    x = x / jnp.maximum(jnp.sqrt(ss), 1e-12)
    out_ref[0] = x[:, :HALF]
    out_ref[1] = x[:, HALF:]


def _prop_body(h0, srcp, dstp, wp, zrows, h0i, h1i, h2,
               acc, src_b, dst_b, w_b, rows16_b, rowsf, semi, semg):
    c = lax.axis_index("c")
    s = lax.axis_index("s")
    coff = c * NP
    ebase = s * EPT
    NB = EPT // EB

    def fetch_idx(j, b, sync):
        base = ebase + j * EB
        if sync:
            pltpu.sync_copy(srcp.at[pl.ds(base, EB)], src_b[b])
            pltpu.sync_copy(dstp.at[pl.ds(base, EB)], dst_b[b])
            pltpu.sync_copy(wp.at[pl.ds(base, EB)], w_b[b])
            return None
        return (
            pltpu.async_copy(srcp.at[pl.ds(base, EB)], src_b[b], semi[b]),
            pltpu.async_copy(dstp.at[pl.ds(base, EB)], dst_b[b], semi[b]),
            pltpu.async_copy(wp.at[pl.ds(base, EB)], w_b[b], semi[b]),
        )

    def offset_src(b):
        for k in range(EB // 16):
            sl = pl.ds(k * 16, 16)
            src_b[b][sl] = src_b[b][sl] + coff

    def start_gather(h_in, b):
        pltpu.async_copy(h_in.at[src_b[b]], rows16_b[b], semg[b])

    def wait_gather(h_in, b):
        pltpu.make_async_copy(h_in.at[src_b[b]], rows16_b[b], semg[b]).wait()

    def scale_scatter(b):
        rows16 = rows16_b[b]
        w_v = w_b[b]

        def scale(g, carry2):
            wvec = w_v[pl.ds(g * 16, 16)]
            for l in range(16):
                e = g * 16 + l
                wval = wvec[l]
                word = rows16[e]
                av = lax.bitcast_convert_type(word << 16, jnp.float32)
                bv = lax.bitcast_convert_type(
                    word & jnp.int32(-65536), jnp.float32)
                rowsf[e, pl.ds(0, 16)] = av * wval
                rowsf[e, pl.ds(16, 16)] = bv * wval
            return carry2

        lax.fori_loop(0, EB // 16, scale, 0)
        pltpu.sync_copy(rowsf, acc.at[dst_b[b]], add=True)

    def pack_rows(n_rows):
        def one(r, carry):
            abits = lax.bitcast_convert_type(rowsf[r, pl.ds(0, 16)], jnp.int32)
            bbits = lax.bitcast_convert_type(rowsf[r, pl.ds(16, 16)], jnp.int32)
            lo = lax.shift_right_logical(abits + jnp.int32(0x8000), 16)
            hi = (bbits + jnp.int32(0x8000)) & jnp.int32(-65536)
            rows16_b[0][r, :] = lo | hi
            return carry

        lax.fori_loop(0, n_rows, one, 0)

    def convert_rows(src_f32, src_off, dst_i32, dst_off):
        def chunk(t, carry):
            pltpu.sync_copy(src_f32.at[pl.ds(src_off + t * CB, CB)],
                            rowsf.at[pl.ds(0, CB)])
            pack_rows(CB)
            pltpu.sync_copy(rows16_b[0].at[pl.ds(0, CB)],
                            dst_i32.at[pl.ds(dst_off + t * CB, CB)])
            return carry

        lax.fori_loop(0, ZR // CB, chunk, 0)

    def run_layer(h_in):
        fetch_idx(0, 0, sync=True)
        offset_src(0)
        start_gather(h_in, 0)

        def half(j, cur, nxt):
            jn = jnp.minimum(j + 1, NB - 1)
            descs = fetch_idx(jn, nxt, sync=False)
            wait_gather(h_in, cur)
            for d_ in descs:
                d_.wait()
            offset_src(nxt)
            start_gather(h_in, nxt)
            scale_scatter(cur)

        def pair(jj, carry):
            half(jj * 2, 0, 1)
            half(jj * 2 + 1, 1, 0)
            return carry

        lax.fori_loop(0, NB // 2, pair, 0)
        wait_gather(h_in, 0)

    convert_rows(h0, coff + s * ZR, h0i, coff + s * ZR)
    pltpu.sync_copy(zrows, acc.at[pl.ds(s * ZR, ZR)])
    plsc.subcore_barrier()
    run_layer(h0i)
    plsc.subcore_barrier()
    convert_rows(acc, s * ZR, h1i, coff + s * ZR)
    pltpu.sync_copy(zrows, acc.at[pl.ds(s * ZR, ZR)])
    plsc.subcore_barrier()
    run_layer(h1i)
    plsc.subcore_barrier()
    pltpu.sync_copy(acc.at[pl.ds(s * ZR, ZR)], h2.at[pl.ds(coff + s * ZR, ZR)])


def kernel(c_feat, t_feat, edge_weight, W1, b1, W2, b2, edge_index):
    w1c = W1[:D]
    w1t = W1[D:]
    b1r = b1.reshape(1, D)
    b2r = b2.reshape(1, D)
    h0 = pl.pallas_call(
        _mlp_body,
        grid=(N // MLP_BN,),
        in_specs=[
            pl.BlockSpec((MLP_BN, D), lambda i: (i, 0)),
            pl.BlockSpec((MLP_BN, D), lambda i: (i, 0)),
            pl.BlockSpec((D, D), lambda i: (0, 0)),
            pl.BlockSpec((D, D), lambda i: (0, 0)),
            pl.BlockSpec((1, D), lambda i: (0, 0)),
            pl.BlockSpec((D, D), lambda i: (0, 0)),
            pl.BlockSpec((1, D), lambda i: (0, 0)),
        ],
        out_specs=pl.BlockSpec((2, MLP_BN, HALF), lambda i: (0, i, 0)),
        out_shape=jax.ShapeDtypeStruct((2, NP, HALF), jnp.float32),
    )(c_feat, t_feat, w1c, w1t, b1r, W2, b2r)
    h0s = h0.reshape(2 * NP, HALF)

    src = edge_index[0].astype(jnp.int32)
    dst = edge_index[1].astype(jnp.int32)
    pad = EPAD - E
    srcp = jnp.concatenate([src, jnp.zeros((pad,), jnp.int32)])
    dstp = jnp.concatenate([dst, jnp.zeros((pad,), jnp.int32)])
    wp = jnp.concatenate([edge_weight, jnp.zeros((pad,), jnp.float32)])
    zrows = jnp.zeros((ZR, HALF), jnp.float32)

    mesh = plsc.VectorSubcoreMesh(core_axis_name="c", subcore_axis_name="s")
    _, _, h2s = pl.kernel(
        _prop_body,
        out_type=(
            jax.ShapeDtypeStruct((2 * NP, 16), jnp.int32),
            jax.ShapeDtypeStruct((2 * NP, 16), jnp.int32),
            jax.ShapeDtypeStruct((2 * NP, HALF), jnp.float32),
        ),
        mesh=mesh,
        compiler_params=pltpu.CompilerParams(use_tc_tiling_on_sc=False),
        scratch_types=[
            pltpu.VMEM_SHARED((NP, HALF), jnp.float32),
            [pltpu.VMEM((EB,), jnp.int32) for _ in range(2)],
            [pltpu.VMEM((EB,), jnp.int32) for _ in range(2)],
            [pltpu.VMEM((EB,), jnp.float32) for _ in range(2)],
            [pltpu.VMEM((EB, 16), jnp.int32) for _ in range(2)],
            pltpu.VMEM((EB, HALF), jnp.float32),
            [pltpu.SemaphoreType.DMA for _ in range(2)],
            [pltpu.SemaphoreType.DMA for _ in range(2)],
        ],
    )(h0s, srcp, dstp, wp, zrows)
    return jnp.concatenate([h2s[:N], h2s[NP:NP + N]], axis=-1)


# speedup vs baseline: 1.0361x; 1.0361x over previous
import jax
import jax.numpy as jnp
from jax import lax
from jax.experimental import pallas as pl
from jax.experimental.pallas import tpu as pltpu
from jax.experimental.pallas import tpu_sc as plsc

N = 50000
D = 64
HALF = 32
E = 800000
NSC = 2
NTILE = 16
EPAD = 819200
EPT = EPAD // NTILE
EB = 400
NP = 50048
ZR = NP // NTILE

MLP_BN = 1000


def _mlp_body(c_ref, t_ref, w1c_ref, w1t_ref, b1_ref, w2_ref, b2_ref, out_ref):
    x = c_ref[...] @ w1c_ref[...] + t_ref[...] @ w1t_ref[...] + b1_ref[...]
    x = jnp.maximum(x, 0.0)
    x = jnp.maximum(x @ w2_ref[...] + b2_ref[...], 0.0)
    ss = jnp.sum(x * x, axis=-1, keepdims=True)
    x = x / jnp.maximum(jnp.sqrt(ss), 1e-12)
    out_ref[0] = x[:, :HALF]
    out_ref[1] = x[:, HALF:]


def _prop_body(h0, pidx, zrows, h1, h2,
               acc, idx_b, rows_b, semi, semg):
    c = lax.axis_index("c")
    s = lax.axis_index("s")
    coff = c * NP
    NB = EPT // EB
    NBT = EPAD // EB

    def fetch_idx(j, b, sync):
        row0 = (c * NBT + s * NB + j) * 3
        if sync:
            pltpu.sync_copy(pidx.at[pl.ds(row0, 3)], idx_b[b])
            return None
        return pltpu.async_copy(pidx.at[pl.ds(row0, 3)], idx_b[b], semi[b])

    def start_gather(h_in, b):
        pltpu.async_copy(h_in.at[idx_b[b].at[0]], rows_b[b], semg[b])

    def wait_gather(h_in, b):
        pltpu.make_async_copy(
            h_in.at[idx_b[b].at[0]], rows_b[b], semg[b]).wait()

    def scale_scatter(b):
        rows_v = rows_b[b]

        def scale(g, carry2):
            wvec = lax.bitcast_convert_type(
                idx_b[b][2, pl.ds(g * 16, 16)], jnp.float32)
            for l in range(16):
                e = g * 16 + l
                wval = wvec[l]
                rows_v[e, pl.ds(0, 16)] = rows_v[e, pl.ds(0, 16)] * wval
                rows_v[e, pl.ds(16, 16)] = rows_v[e, pl.ds(16, 16)] * wval
            return carry2

        lax.fori_loop(0, EB // 16, scale, 0)
        pltpu.sync_copy(rows_v, acc.at[idx_b[b].at[1]], add=True)

    def run_layer(h_in):
        fetch_idx(0, 0, sync=True)
        start_gather(h_in, 0)

        def half(j, cur, nxt):
            jn = jnp.minimum(j + 1, NB - 1)
            desc = fetch_idx(jn, nxt, sync=False)
            wait_gather(h_in, cur)
            desc.wait()
            start_gather(h_in, nxt)
            scale_scatter(cur)

        def pair(jj, carry):
            half(jj * 2, 0, 1)
            half(jj * 2 + 1, 1, 0)
            return carry

        lax.fori_loop(0, NB // 2, pair, 0)
        wait_gather(h_in, 0)

    pltpu.sync_copy(zrows, acc.at[pl.ds(s * ZR, ZR)])
    plsc.subcore_barrier()
    run_layer(h0)
    plsc.subcore_barrier()
    pltpu.sync_copy(acc.at[pl.ds(s * ZR, ZR)], h1.at[pl.ds(coff + s * ZR, ZR)])
    pltpu.sync_copy(zrows, acc.at[pl.ds(s * ZR, ZR)])
    plsc.subcore_barrier()
    run_layer(h1)
    plsc.subcore_barrier()
    pltpu.sync_copy(acc.at[pl.ds(s * ZR, ZR)], h2.at[pl.ds(coff + s * ZR, ZR)])


def kernel(c_feat, t_feat, edge_weight, W1, b1, W2, b2, edge_index):
    w1c = W1[:D]
    w1t = W1[D:]
    b1r = b1.reshape(1, D)
    b2r = b2.reshape(1, D)
    h0 = pl.pallas_call(
        _mlp_body,
        grid=(N // MLP_BN,),
        in_specs=[
            pl.BlockSpec((MLP_BN, D), lambda i: (i, 0)),
            pl.BlockSpec((MLP_BN, D), lambda i: (i, 0)),
            pl.BlockSpec((D, D), lambda i: (0, 0)),
            pl.BlockSpec((D, D), lambda i: (0, 0)),
            pl.BlockSpec((1, D), lambda i: (0, 0)),
            pl.BlockSpec((D, D), lambda i: (0, 0)),
            pl.BlockSpec((1, D), lambda i: (0, 0)),
        ],
        out_specs=pl.BlockSpec((2, MLP_BN, HALF), lambda i: (0, i, 0)),
        out_shape=jax.ShapeDtypeStruct((2, NP, HALF), jnp.float32),
    )(c_feat, t_feat, w1c, w1t, b1r, W2, b2r)
    h0s = h0.reshape(2 * NP, HALF)

    src = edge_index[0].astype(jnp.int32)
    dst = edge_index[1].astype(jnp.int32)
    pad = EPAD - E
    srcp = jnp.concatenate([src, jnp.zeros((pad,), jnp.int32)])
    dstp = jnp.concatenate([dst, jnp.zeros((pad,), jnp.int32)])
    wp = jnp.concatenate([edge_weight, jnp.zeros((pad,), jnp.float32)])
    nbt = EPAD // EB
    srcb = srcp.reshape(nbt, EB)
    dstb = dstp.reshape(nbt, EB)
    wbits = jax.lax.bitcast_convert_type(wp, jnp.int32).reshape(nbt, EB)
    pidx = jnp.stack(
        [jnp.stack([srcb, dstb, wbits], axis=1),
         jnp.stack([srcb + NP, dstb, wbits], axis=1)],
        axis=0).reshape(2 * nbt * 3, EB)
    zrows = jnp.zeros((ZR, HALF), jnp.float32)

    mesh = plsc.VectorSubcoreMesh(core_axis_name="c", subcore_axis_name="s")
    _, h2s = pl.kernel(
        _prop_body,
        out_type=(
            jax.ShapeDtypeStruct((2 * NP, HALF), jnp.float32),
            jax.ShapeDtypeStruct((2 * NP, HALF), jnp.float32),
        ),
        mesh=mesh,
        compiler_params=pltpu.CompilerParams(use_tc_tiling_on_sc=False),
        scratch_types=[
            pltpu.VMEM_SHARED((NP, HALF), jnp.float32),
            [pltpu.VMEM((3, EB), jnp.int32) for _ in range(2)],
            [pltpu.VMEM((EB, HALF), jnp.float32) for _ in range(2)],
            [pltpu.SemaphoreType.DMA for _ in range(2)],
            [pltpu.SemaphoreType.DMA for _ in range(2)],
        ],
    )(h0s, pidx, zrows)
    return jnp.concatenate([h2s[:N], h2s[NP:NP + N]], axis=-1)


# speedup vs baseline: 1.0886x; 1.0506x over previous
import jax
import jax.numpy as jnp
from jax import lax
from jax.experimental import pallas as pl
from jax.experimental.pallas import tpu as pltpu
from jax.experimental.pallas import tpu_sc as plsc

N = 50000
D = 64
HALF = 32
E = 800000
NSC = 2
NTILE = 16
EPAD = 819200
EPT = EPAD // NTILE
EB = 400
NP = 50048
ZR = NP // NTILE

MLP_BN = 1000


def _mlp_body(c_ref, t_ref, w1c_ref, w1t_ref, b1_ref, w2_ref, b2_ref, out_ref):
    x = c_ref[...] @ w1c_ref[...] + t_ref[...] @ w1t_ref[...] + b1_ref[...]
    x = jnp.maximum(x, 0.0)
    x = jnp.maximum(x @ w2_ref[...] + b2_ref[...], 0.0)
    ss = jnp.sum(x * x, axis=-1, keepdims=True)
    x = x / jnp.maximum(jnp.sqrt(ss), 1e-12)
    out_ref[0] = x[:, :HALF]
    out_ref[1] = x[:, HALF:]


def _prop_body(h0, pidx, zrows, h1, h2,
               acc, idx_b, rows_b, semi, semg):
    c = lax.axis_index("c")
    s = lax.axis_index("s")
    coff = c * NP
    NB = EPT // EB
    NBT = EPAD // EB

    def fetch_idx(j, b, sync):
        row0 = (c * NBT + s * NB + j) * 3
        if sync:
            pltpu.sync_copy(pidx.at[pl.ds(row0, 3)], idx_b[b])
            return None
        return pltpu.async_copy(pidx.at[pl.ds(row0, 3)], idx_b[b], semi[b])

    def wait_idx(b):
        pltpu.make_async_copy(pidx.at[pl.ds(0, 3)], idx_b[b], semi[b]).wait()

    def start_gather(h_in, r, b):
        pltpu.async_copy(h_in.at[idx_b[b].at[0]], rows_b[r], semg[r])

    def wait_gather(h_in, r):
        pltpu.make_async_copy(
            h_in.at[idx_b[0].at[0]], rows_b[r], semg[r]).wait()

    def scale_scatter(r, b):
        rows_v = rows_b[r]

        def scale(g, carry2):
            wvec = lax.bitcast_convert_type(
                idx_b[b][2, pl.ds(g * 16, 16)], jnp.float32)
            for l in range(16):
                e = g * 16 + l
                wval = wvec[l]
                rows_v[e, pl.ds(0, 16)] = rows_v[e, pl.ds(0, 16)] * wval
                rows_v[e, pl.ds(16, 16)] = rows_v[e, pl.ds(16, 16)] * wval
            return carry2

        lax.fori_loop(0, EB // 16, scale, 0)
        pltpu.sync_copy(rows_v, acc.at[idx_b[b].at[1]], add=True)

    def run_layer(h_in):
        fetch_idx(0, 0, sync=True)
        fetch_idx(1, 1, sync=False)
        start_gather(h_in, 0, 0)

        def half(j, i0, i1, i2, r0, r1):
            fetch_idx(jnp.minimum(j + 2, NB - 1), i2, sync=False)
            wait_idx(i1)
            wait_gather(h_in, r0)
            start_gather(h_in, r1, i1)
            scale_scatter(r0, i0)

        def quad(qq, carry):
            j0 = qq * 4
            half(j0 + 0, 0, 1, 2, 0, 1)
            half(j0 + 1, 1, 2, 3, 1, 0)
            half(j0 + 2, 2, 3, 0, 0, 1)
            half(j0 + 3, 3, 0, 1, 1, 0)
            return carry

        lax.fori_loop(0, NB // 4, quad, 0)
        wait_idx(1)
        wait_gather(h_in, 0)

    pltpu.sync_copy(zrows, acc.at[pl.ds(s * ZR, ZR)])
    plsc.subcore_barrier()
    run_layer(h0)
    plsc.subcore_barrier()
    pltpu.sync_copy(acc.at[pl.ds(s * ZR, ZR)], h1.at[pl.ds(coff + s * ZR, ZR)])
    pltpu.sync_copy(zrows, acc.at[pl.ds(s * ZR, ZR)])
    plsc.subcore_barrier()
    run_layer(h1)
    plsc.subcore_barrier()
    pltpu.sync_copy(acc.at[pl.ds(s * ZR, ZR)], h2.at[pl.ds(coff + s * ZR, ZR)])


def kernel(c_feat, t_feat, edge_weight, W1, b1, W2, b2, edge_index):
    w1c = W1[:D]
    w1t = W1[D:]
    b1r = b1.reshape(1, D)
    b2r = b2.reshape(1, D)
    h0 = pl.pallas_call(
        _mlp_body,
        grid=(N // MLP_BN,),
        in_specs=[
            pl.BlockSpec((MLP_BN, D), lambda i: (i, 0)),
            pl.BlockSpec((MLP_BN, D), lambda i: (i, 0)),
            pl.BlockSpec((D, D), lambda i: (0, 0)),
            pl.BlockSpec((D, D), lambda i: (0, 0)),
            pl.BlockSpec((1, D), lambda i: (0, 0)),
            pl.BlockSpec((D, D), lambda i: (0, 0)),
            pl.BlockSpec((1, D), lambda i: (0, 0)),
        ],
        out_specs=pl.BlockSpec((2, MLP_BN, HALF), lambda i: (0, i, 0)),
        out_shape=jax.ShapeDtypeStruct((2, NP, HALF), jnp.float32),
    )(c_feat, t_feat, w1c, w1t, b1r, W2, b2r)
    h0s = h0.reshape(2 * NP, HALF)

    src = edge_index[0].astype(jnp.int32)
    dst = edge_index[1].astype(jnp.int32)
    pad = EPAD - E
    srcp = jnp.concatenate([src, jnp.zeros((pad,), jnp.int32)])
    dstp = jnp.concatenate([dst, jnp.zeros((pad,), jnp.int32)])
    wp = jnp.concatenate([edge_weight, jnp.zeros((pad,), jnp.float32)])
    nbt = EPAD // EB
    srcb = srcp.reshape(nbt, EB)
    dstb = dstp.reshape(nbt, EB)
    wbits = jax.lax.bitcast_convert_type(wp, jnp.int32).reshape(nbt, EB)
    pidx = jnp.stack(
        [jnp.stack([srcb, dstb, wbits], axis=1),
         jnp.stack([srcb + NP, dstb, wbits], axis=1)],
        axis=0).reshape(2 * nbt * 3, EB)
    zrows = jnp.zeros((ZR, HALF), jnp.float32)

    mesh = plsc.VectorSubcoreMesh(core_axis_name="c", subcore_axis_name="s")
    _, h2s = pl.kernel(
        _prop_body,
        out_type=(
            jax.ShapeDtypeStruct((2 * NP, HALF), jnp.float32),
            jax.ShapeDtypeStruct((2 * NP, HALF), jnp.float32),
        ),
        mesh=mesh,
        compiler_params=pltpu.CompilerParams(use_tc_tiling_on_sc=False),
        scratch_types=[
            pltpu.VMEM_SHARED((NP, HALF), jnp.float32),
            [pltpu.VMEM((3, EB), jnp.int32) for _ in range(4)],
            [pltpu.VMEM((EB, HALF), jnp.float32) for _ in range(2)],
            [pltpu.SemaphoreType.DMA for _ in range(4)],
            [pltpu.SemaphoreType.DMA for _ in range(2)],
        ],
    )(h0s, pidx, zrows)
    return jnp.concatenate([h2s[:N], h2s[NP:NP + N]], axis=-1)


# speedup vs baseline: 1.1311x; 1.0390x over previous
import jax
import jax.numpy as jnp
from jax import lax
from jax.experimental import pallas as pl
from jax.experimental.pallas import tpu as pltpu
from jax.experimental.pallas import tpu_sc as plsc

N = 50000
D = 64
HALF = 32
E = 800000
NSC = 2
NTILE = 16
EPAD = 819200
EPT = EPAD // NTILE
EB = 400
NP = 50048
ZR = NP // NTILE

MLP_BN = 1000


def _mlp_body(c_ref, t_ref, w1c_ref, w1t_ref, b1_ref, w2_ref, b2_ref, out_ref):
    x = c_ref[...] @ w1c_ref[...] + t_ref[...] @ w1t_ref[...] + b1_ref[...]
    x = jnp.maximum(x, 0.0)
    x = jnp.maximum(x @ w2_ref[...] + b2_ref[...], 0.0)
    ss = jnp.sum(x * x, axis=-1, keepdims=True)
    x = x / jnp.maximum(jnp.sqrt(ss), 1e-12)
    out_ref[0] = x[:, :HALF]
    out_ref[1] = x[:, HALF:]


def _prop_body(h0, pidx, zrows, h1, h2,
               acc, idx_b, rows_b, semi, semg):
    c = lax.axis_index("c")
    s = lax.axis_index("s")
    coff = c * NP
    NB = EPT // EB
    NBT = EPAD // EB

    def fetch_idx(j, b, sync):
        row0 = (c * NBT + s * NB + j) * 3
        if sync:
            pltpu.sync_copy(pidx.at[pl.ds(row0, 3)], idx_b[b])
            return None
        return pltpu.async_copy(pidx.at[pl.ds(row0, 3)], idx_b[b], semi[b])

    def wait_idx(b):
        pltpu.make_async_copy(pidx.at[pl.ds(0, 3)], idx_b[b], semi[b]).wait()

    def start_gather(h_in, r, b):
        pltpu.async_copy(h_in.at[idx_b[b].at[0]], rows_b[r], semg[r])

    def wait_gather(h_in, r):
        pltpu.make_async_copy(
            h_in.at[idx_b[0].at[0]], rows_b[r], semg[r]).wait()

    def scale_scatter(r, b):
        rows_v = rows_b[r]

        def scale(g, carry2):
            wvec = lax.bitcast_convert_type(
                idx_b[b][2, pl.ds(g * 16, 16)], jnp.float32)
            for l in range(16):
                e = g * 16 + l
                wval = wvec[l]
                rows_v[e, pl.ds(0, 16)] = rows_v[e, pl.ds(0, 16)] * wval
                rows_v[e, pl.ds(16, 16)] = rows_v[e, pl.ds(16, 16)] * wval
            return carry2

        lax.fori_loop(0, EB // 16, scale, 0)
        pltpu.sync_copy(rows_v, acc.at[idx_b[b].at[1]], add=True)

    def run_layer(h_in):
        fetch_idx(0, 0, sync=True)
        fetch_idx(1, 1, sync=False)
        start_gather(h_in, 0, 0)

        def half(j, i0, i1, i2, r0, r1):
            fetch_idx(jnp.minimum(j + 2, NB - 1), i2, sync=False)
            wait_idx(i1)
            start_gather(h_in, r1, i1)
            wait_gather(h_in, r0)
            scale_scatter(r0, i0)

        def quad(qq, carry):
            j0 = qq * 4
            half(j0 + 0, 0, 1, 2, 0, 1)
            half(j0 + 1, 1, 2, 3, 1, 0)
            half(j0 + 2, 2, 3, 0, 0, 1)
            half(j0 + 3, 3, 0, 1, 1, 0)
            return carry

        lax.fori_loop(0, NB // 4, quad, 0)
        wait_idx(1)
        wait_gather(h_in, 0)

    pltpu.sync_copy(zrows, acc.at[pl.ds(s * ZR, ZR)])
    plsc.subcore_barrier()
    run_layer(h0)
    plsc.subcore_barrier()
    pltpu.sync_copy(acc.at[pl.ds(s * ZR, ZR)], h1.at[pl.ds(coff + s * ZR, ZR)])
    pltpu.sync_copy(zrows, acc.at[pl.ds(s * ZR, ZR)])
    plsc.subcore_barrier()
    run_layer(h1)
    plsc.subcore_barrier()
    pltpu.sync_copy(acc.at[pl.ds(s * ZR, ZR)], h2.at[pl.ds(coff + s * ZR, ZR)])


def kernel(c_feat, t_feat, edge_weight, W1, b1, W2, b2, edge_index):
    w1c = W1[:D]
    w1t = W1[D:]
    b1r = b1.reshape(1, D)
    b2r = b2.reshape(1, D)
    h0 = pl.pallas_call(
        _mlp_body,
        grid=(N // MLP_BN,),
        in_specs=[
            pl.BlockSpec((MLP_BN, D), lambda i: (i, 0)),
            pl.BlockSpec((MLP_BN, D), lambda i: (i, 0)),
            pl.BlockSpec((D, D), lambda i: (0, 0)),
            pl.BlockSpec((D, D), lambda i: (0, 0)),
            pl.BlockSpec((1, D), lambda i: (0, 0)),
            pl.BlockSpec((D, D), lambda i: (0, 0)),
            pl.BlockSpec((1, D), lambda i: (0, 0)),
        ],
        out_specs=pl.BlockSpec((2, MLP_BN, HALF), lambda i: (0, i, 0)),
        out_shape=jax.ShapeDtypeStruct((2, NP, HALF), jnp.float32),
    )(c_feat, t_feat, w1c, w1t, b1r, W2, b2r)
    h0s = h0.reshape(2 * NP, HALF)

    src = edge_index[0].astype(jnp.int32)
    dst = edge_index[1].astype(jnp.int32)
    pad = EPAD - E
    srcp = jnp.concatenate([src, jnp.zeros((pad,), jnp.int32)])
    dstp = jnp.concatenate([dst, jnp.zeros((pad,), jnp.int32)])
    wp = jnp.concatenate([edge_weight, jnp.zeros((pad,), jnp.float32)])
    nbt = EPAD // EB
    srcb = srcp.reshape(nbt, EB)
    dstb = dstp.reshape(nbt, EB)
    wbits = jax.lax.bitcast_convert_type(wp, jnp.int32).reshape(nbt, EB)
    pidx = jnp.stack(
        [jnp.stack([srcb, dstb, wbits], axis=1),
         jnp.stack([srcb + NP, dstb, wbits], axis=1)],
        axis=0).reshape(2 * nbt * 3, EB)
    zrows = jnp.zeros((ZR, HALF), jnp.float32)

    mesh = plsc.VectorSubcoreMesh(core_axis_name="c", subcore_axis_name="s")
    _, h2s = pl.kernel(
        _prop_body,
        out_type=(
            jax.ShapeDtypeStruct((2 * NP, HALF), jnp.float32),
            jax.ShapeDtypeStruct((2 * NP, HALF), jnp.float32),
        ),
        mesh=mesh,
        compiler_params=pltpu.CompilerParams(use_tc_tiling_on_sc=False),
        scratch_types=[
            pltpu.VMEM_SHARED((NP, HALF), jnp.float32),
            [pltpu.VMEM((3, EB), jnp.int32) for _ in range(4)],
            [pltpu.VMEM((EB, HALF), jnp.float32) for _ in range(2)],
            [pltpu.SemaphoreType.DMA for _ in range(4)],
            [pltpu.SemaphoreType.DMA for _ in range(2)],
        ],
    )(h0s, pidx, zrows)
    return jnp.concatenate([h2s[:N], h2s[NP:NP + N]], axis=-1)


# speedup vs baseline: 1.1349x; 1.0034x over previous
import jax
import jax.numpy as jnp
from jax import lax
from jax.experimental import pallas as pl
from jax.experimental.pallas import tpu as pltpu
from jax.experimental.pallas import tpu_sc as plsc

N = 50000
D = 64
HALF = 32
E = 800000
NSC = 2
NTILE = 16
EPAD = 819200
EPT = EPAD // NTILE
EB = 400
NP = 50048
ZR = NP // NTILE

MLP_BN = 2000


def _mlp_body(c_ref, t_ref, w1c_ref, w1t_ref, b1_ref, w2_ref, b2_ref, out_ref):
    x = c_ref[...] @ w1c_ref[...] + t_ref[...] @ w1t_ref[...] + b1_ref[...]
    x = jnp.maximum(x, 0.0)
    x = jnp.maximum(x @ w2_ref[...] + b2_ref[...], 0.0)
    ss = jnp.sum(x * x, axis=-1, keepdims=True)
    x = x / jnp.maximum(jnp.sqrt(ss), 1e-12)
    out_ref[0] = x[:, :HALF]
    out_ref[1] = x[:, HALF:]


def _prop_body(h0, srcp, dstp, wp, zrows, h1, h2,
               acc, src_b, dst_b, w_b, rows_b, semi, semg):
    c = lax.axis_index("c")
    s = lax.axis_index("s")
    coff = c * NP
    ebase = s * EPT
    NB = EPT // EB

    def fetch_idx(j, b, sync):
        base = ebase + j * EB
        if sync:
            pltpu.sync_copy(srcp.at[pl.ds(base, EB)], src_b[b])
            pltpu.sync_copy(dstp.at[pl.ds(base, EB)], dst_b[b])
            pltpu.sync_copy(wp.at[pl.ds(base, EB)], w_b[b])
            return
        pltpu.async_copy(srcp.at[pl.ds(base, EB)], src_b[b], semi[b])
        pltpu.async_copy(dstp.at[pl.ds(base, EB)], dst_b[b], semi[b])
        pltpu.async_copy(wp.at[pl.ds(base, EB)], w_b[b], semi[b])

    def wait_idx(b):
        pltpu.make_async_copy(srcp.at[pl.ds(0, EB)], src_b[b], semi[b]).wait()
        pltpu.make_async_copy(dstp.at[pl.ds(0, EB)], dst_b[b], semi[b]).wait()
        pltpu.make_async_copy(wp.at[pl.ds(0, EB)], w_b[b], semi[b]).wait()

    def offset_src(b):
        for k in range(EB // 16):
            sl = pl.ds(k * 16, 16)
            src_b[b][sl] = src_b[b][sl] + coff

    def start_gather(h_in, r, b):
        pltpu.async_copy(h_in.at[src_b[b]], rows_b[r], semg[r])

    def wait_gather(h_in, r):
        pltpu.make_async_copy(
            h_in.at[src_b[0]], rows_b[r], semg[r]).wait()

    def scale_scatter(r, b):
        rows_v = rows_b[r]
        w_v = w_b[b]

        def scale(g, carry2):
            wvec = w_v[pl.ds(g * 16, 16)]
            for l in range(16):
                e = g * 16 + l
                wval = wvec[l]
                rows_v[e, pl.ds(0, 16)] = rows_v[e, pl.ds(0, 16)] * wval
                rows_v[e, pl.ds(16, 16)] = rows_v[e, pl.ds(16, 16)] * wval
            return carry2

        lax.fori_loop(0, EB // 16, scale, 0)
        pltpu.sync_copy(rows_v, acc.at[dst_b[b]], add=True)

    def run_layer(h_in):
        fetch_idx(0, 0, sync=True)
        fetch_idx(1, 1, sync=False)
        offset_src(0)
        start_gather(h_in, 0, 0)

        def half(j, i0, i1, i2, r0, r1):
            fetch_idx(jnp.minimum(j + 2, NB - 1), i2, sync=False)
            wait_idx(i1)
            offset_src(i1)
            start_gather(h_in, r1, i1)
            wait_gather(h_in, r0)
            scale_scatter(r0, i0)

        def quad(qq, carry):
            j0 = qq * 4
            half(j0 + 0, 0, 1, 2, 0, 1)
            half(j0 + 1, 1, 2, 3, 1, 0)
            half(j0 + 2, 2, 3, 0, 0, 1)
            half(j0 + 3, 3, 0, 1, 1, 0)
            return carry

        lax.fori_loop(0, NB // 4, quad, 0)
        wait_idx(1)
        wait_gather(h_in, 0)

    pltpu.sync_copy(zrows, acc.at[pl.ds(s * ZR, ZR)])
    plsc.subcore_barrier()
    run_layer(h0)
    plsc.subcore_barrier()
    pltpu.sync_copy(acc.at[pl.ds(s * ZR, ZR)], h1.at[pl.ds(coff + s * ZR, ZR)])
    pltpu.sync_copy(zrows, acc.at[pl.ds(s * ZR, ZR)])
    plsc.subcore_barrier()
    run_layer(h1)
    plsc.subcore_barrier()
    pltpu.sync_copy(acc.at[pl.ds(s * ZR, ZR)], h2.at[pl.ds(coff + s * ZR, ZR)])


def kernel(c_feat, t_feat, edge_weight, W1, b1, W2, b2, edge_index):
    w1c = W1[:D]
    w1t = W1[D:]
    b1r = b1.reshape(1, D)
    b2r = b2.reshape(1, D)
    h0 = pl.pallas_call(
        _mlp_body,
        grid=(N // MLP_BN,),
        in_specs=[
            pl.BlockSpec((MLP_BN, D), lambda i: (i, 0)),
            pl.BlockSpec((MLP_BN, D), lambda i: (i, 0)),
            pl.BlockSpec((D, D), lambda i: (0, 0)),
            pl.BlockSpec((D, D), lambda i: (0, 0)),
            pl.BlockSpec((1, D), lambda i: (0, 0)),
            pl.BlockSpec((D, D), lambda i: (0, 0)),
            pl.BlockSpec((1, D), lambda i: (0, 0)),
        ],
        out_specs=pl.BlockSpec((2, MLP_BN, HALF), lambda i: (0, i, 0)),
        out_shape=jax.ShapeDtypeStruct((2, NP, HALF), jnp.float32),
    )(c_feat, t_feat, w1c, w1t, b1r, W2, b2r)
    h0s = h0.reshape(2 * NP, HALF)

    src = edge_index[0].astype(jnp.int32)
    dst = edge_index[1].astype(jnp.int32)
    pad = EPAD - E
    srcp = jnp.concatenate([src, jnp.zeros((pad,), jnp.int32)])
    dstp = jnp.concatenate([dst, jnp.zeros((pad,), jnp.int32)])
    wp = jnp.concatenate([edge_weight, jnp.zeros((pad,), jnp.float32)])
    zrows = jnp.zeros((ZR, HALF), jnp.float32)

    mesh = plsc.VectorSubcoreMesh(core_axis_name="c", subcore_axis_name="s")
    _, h2s = pl.kernel(
        _prop_body,
        out_type=(
            jax.ShapeDtypeStruct((2 * NP, HALF), jnp.float32),
            jax.ShapeDtypeStruct((2 * NP, HALF), jnp.float32),
        ),
        mesh=mesh,
        compiler_params=pltpu.CompilerParams(use_tc_tiling_on_sc=False),
        scratch_types=[
            pltpu.VMEM_SHARED((NP, HALF), jnp.float32),
            [pltpu.VMEM((EB,), jnp.int32) for _ in range(4)],
            [pltpu.VMEM((EB,), jnp.int32) for _ in range(4)],
            [pltpu.VMEM((EB,), jnp.float32) for _ in range(4)],
            [pltpu.VMEM((EB, HALF), jnp.float32) for _ in range(2)],
            [pltpu.SemaphoreType.DMA for _ in range(4)],
            [pltpu.SemaphoreType.DMA for _ in range(2)],
        ],
    )(h0s, srcp, dstp, wp, zrows)
    return jnp.concatenate([h2s[:N], h2s[NP:NP + N]], axis=-1)
